# baseline (speedup 1.0000x reference)
import functools

import jax
import jax.numpy as jnp
from jax import lax
from jax.experimental import pallas as pl
from jax.experimental.pallas import tpu as pltpu
from jax.experimental.pallas import tpu_sc as plsc

_N = 10000
_E = 320000
_D_IN = 128
_H1 = 64
_H2 = 128
_D_OUT = 256
_B = 64

_NC = 2
_NS = 16
_NW = _NC * _NS
_EBLK = 128
_GRP = 4
_E_PER_TILE = 10240
_NBLK = _E_PER_TILE // _EBLK
_NGRP = _NBLK // _GRP
_E_PAD = _E_PER_TILE * _NW
_NPAD = 10048
_RPT = _NPAD // _NS
_CW = 8

_R = 1000
_GRID = _N // _R


def _sc_mesh():
    return plsc.VectorSubcoreMesh(core_axis_name="c", subcore_axis_name="s")


@functools.lru_cache(maxsize=None)
def _deg_kernel():
    @functools.partial(
        pl.kernel,
        out_type=jax.ShapeDtypeStruct((_NC, _NPAD, _CW), jnp.float32),
        mesh=_sc_mesh(),
        compiler_params=pltpu.CompilerParams(use_tc_tiling_on_sc=False),
        scratch_types=[
            pltpu.VMEM((_NBLK, _EBLK), jnp.int32),
            pltpu.VMEM((_EBLK, _CW), jnp.float32),
            pltpu.VMEM_SHARED((_NPAD, _CW), jnp.float32),
        ],
    )
    def k(dst2d_hbm, ones_hbm, zero_hbm, out_hbm, dstbuf, ones_v, degS):
        c = lax.axis_index("c")
        s = lax.axis_index("s")
        w = s * _NC + c
        r0 = s * _RPT
        pltpu.sync_copy(zero_hbm.at[pl.ds(r0, _RPT)],
                        degS.at[pl.ds(r0, _RPT)])
        pltpu.sync_copy(dst2d_hbm.at[pl.ds(w * _NBLK, _NBLK)], dstbuf)
        pltpu.sync_copy(ones_hbm, ones_v)
        plsc.subcore_barrier()

        def blk(j, carry):
            pltpu.sync_copy(ones_v, degS.at[dstbuf.at[j]], add=True)
            return carry

        lax.fori_loop(0, _NBLK, blk, 0)
        plsc.subcore_barrier()
        pltpu.sync_copy(degS.at[pl.ds(r0, _RPT)],
                        out_hbm.at[c, pl.ds(r0, _RPT)])

    return k


@functools.lru_cache(maxsize=None)
def _edge_scatter_kernel(d):
    rows_per_buf = _GRP * _EBLK

    @functools.partial(
        pl.kernel,
        out_type=jax.ShapeDtypeStruct((_NC, _NPAD, d), jnp.float32),
        mesh=_sc_mesh(),
        compiler_params=pltpu.CompilerParams(use_tc_tiling_on_sc=False),
        scratch_types=[
            pltpu.VMEM((_E_PER_TILE,), jnp.int32),
            pltpu.VMEM((_NBLK, _EBLK), jnp.int32),
            pltpu.VMEM((rows_per_buf, d), jnp.float32),
            pltpu.VMEM((rows_per_buf, d), jnp.float32),
            pltpu.VMEM_SHARED((_NPAD, d), jnp.float32),
            pltpu.SemaphoreType.DMA,
            pltpu.SemaphoreType.DMA,
            pltpu.SemaphoreType.DMA,
            pltpu.SemaphoreType.DMA,
        ],
    )
    def k(src_hbm, dst2d_hbm, h_hbm, zero_hbm, out_hbm,
          srcflat, dstbuf, rows0, rows1, accS, gsem0, gsem1, ssem0, ssem1):
        c = lax.axis_index("c")
        s = lax.axis_index("s")
        w = s * _NC + c
        r0 = s * _RPT
        pltpu.sync_copy(zero_hbm.at[pl.ds(r0, _RPT)],
                        accS.at[pl.ds(r0, _RPT)])
        pltpu.sync_copy(src_hbm.at[pl.ds(w * _E_PER_TILE, _E_PER_TILE)],
                        srcflat)
        pltpu.sync_copy(dst2d_hbm.at[pl.ds(w * _NBLK, _NBLK)], dstbuf)
        plsc.subcore_barrier()

        def gather_grp(grp, buf, sem):
            for kk in range(_GRP):
                idx = srcflat.at[pl.ds((grp * _GRP + kk) * _EBLK, _EBLK)]
                pltpu.async_copy(h_hbm.at[idx],
                                 buf.at[pl.ds(kk * _EBLK, _EBLK)], sem)

        def buf_wait(buf, sem):
            pltpu.make_async_copy(
                zero_hbm.at[pl.ds(0, rows_per_buf)], buf, sem).wait()

        def scat_grp(grp, buf, sem):
            for kk in range(_GRP):
                pltpu.async_copy(buf.at[pl.ds(kk * _EBLK, _EBLK)],
                                 accS.at[dstbuf.at[grp * _GRP + kk]], sem,
                                 add=True)

        gather_grp(0, rows0, gsem0)
        gather_grp(1, rows1, gsem1)

        def body(g, carry):
            for kk, (buf, gsem, ssem) in enumerate(
                    ((rows0, gsem0, ssem0), (rows1, gsem1, ssem1))):
                grp = 2 * g + kk
                buf_wait(buf, gsem)
                scat_grp(grp, buf, ssem)
                buf_wait(buf, ssem)
                gather_grp(jnp.minimum(grp + 2, _NGRP - 1), buf, gsem)
            return carry

        lax.fori_loop(0, _NGRP // 2, body, 0)
        buf_wait(rows0, gsem0)
        buf_wait(rows1, gsem1)
        plsc.subcore_barrier()
        pltpu.sync_copy(accS.at[pl.ds(r0, _RPT)],
                        out_hbm.at[c, pl.ds(r0, _RPT)])

    return k


def _dinv(cnt_blk):
    return lax.rsqrt(1.0 + cnt_blk[0][:, 0:1] + cnt_blk[1][:, 0:1])


def _k1_body(x_ref, w_ref, cnt_ref, o_ref):
    g = jnp.dot(x_ref[...], w_ref[...], preferred_element_type=jnp.float32)
    o_ref[...] = g * _dinv(cnt_ref)


def _k2_body(acc_ref, h_ref, cnt_ref, b_ref, w2_ref, oa_ref, ob_ref):
    dinv = _dinv(cnt_ref)
    t = jnp.maximum(dinv * (acc_ref[0] + acc_ref[1] + h_ref[...]) + b_ref[...],
                    0.0)
    h2 = jnp.dot(t, w2_ref[...], preferred_element_type=jnp.float32) * dinv
    oa_ref[...] = h2[:, :_H1]
    ob_ref[...] = h2[:, _H1:]


def _k3_body(acca_ref, accb_ref, ha_ref, hb_ref, cnt_ref, b_ref, batch_ref,
             wfc_ref, bfc_ref, o_ref, pooled, counts):
    i = pl.program_id(0)

    @pl.when(i == 0)
    def _():
        pooled[...] = jnp.zeros_like(pooled)
        counts[...] = jnp.zeros_like(counts)

    dinv = _dinv(cnt_ref)
    pre = jnp.concatenate(
        [acca_ref[0] + acca_ref[1] + ha_ref[...],
         accb_ref[0] + accb_ref[1] + hb_ref[...]], axis=1)
    out2 = jnp.maximum(dinv * pre + b_ref[...], 0.0)
    oh = (batch_ref[...] == lax.broadcasted_iota(jnp.int32, (1, _B), 1)
          ).astype(jnp.float32)
    cdims = (((0,), (0,)), ((), ()))
    pooled[...] += lax.dot_general(oh, out2, cdims,
                                   preferred_element_type=jnp.float32)
    counts[...] += lax.dot_general(oh, jnp.ones((_R, 1), jnp.float32), cdims,
                                   preferred_element_type=jnp.float32)

    @pl.when(i == _GRID - 1)
    def _():
        pm = pooled[...] / jnp.maximum(counts[...], 1.0)
        o_ref[...] = jnp.maximum(
            jnp.dot(pm, wfc_ref[...], preferred_element_type=jnp.float32)
            + bfc_ref[...], 0.0)


def _row_spec(d):
    return pl.BlockSpec((_R, d), lambda i: (i, 0))


def _full_spec(shape):
    return pl.BlockSpec(shape, lambda i: tuple(0 for _ in shape))


def _cnt_spec():
    return pl.BlockSpec((_NC, _R, _CW), lambda i: (0, i, 0))


def _acc_spec(d):
    return pl.BlockSpec((_NC, _R, d), lambda i: (0, i, 0))


@functools.lru_cache(maxsize=None)
def _k1_call():
    return pl.pallas_call(
        _k1_body,
        grid=(_GRID,),
        in_specs=[_row_spec(_D_IN), _full_spec((_D_IN, _H1)), _cnt_spec()],
        out_specs=_row_spec(_H1),
        out_shape=jax.ShapeDtypeStruct((_N, _H1), jnp.float32),
    )


@functools.lru_cache(maxsize=None)
def _k2_call():
    return pl.pallas_call(
        _k2_body,
        grid=(_GRID,),
        in_specs=[_acc_spec(_H1), _row_spec(_H1), _cnt_spec(),
                  _full_spec((1, _H1)), _full_spec((_H1, _H2))],
        out_specs=[_row_spec(_H1), _row_spec(_H1)],
        out_shape=[jax.ShapeDtypeStruct((_N, _H1), jnp.float32),
                   jax.ShapeDtypeStruct((_N, _H1), jnp.float32)],
    )


@functools.lru_cache(maxsize=None)
def _k3_call():
    return pl.pallas_call(
        _k3_body,
        grid=(_GRID,),
        in_specs=[_acc_spec(_H1), _acc_spec(_H1), _row_spec(_H1),
                  _row_spec(_H1), _cnt_spec(),
                  _full_spec((1, _H2)), _row_spec(1),
                  _full_spec((_H2, _D_OUT)), _full_spec((1, _D_OUT))],
        out_specs=_full_spec((_B, _D_OUT)),
        out_shape=jax.ShapeDtypeStruct((_B, _D_OUT), jnp.float32),
        scratch_shapes=[pltpu.VMEM((_B, _H2), jnp.float32),
                        pltpu.VMEM((_B, 1), jnp.float32)],
    )


def kernel(x, edge_index, batch, W1, b1, W2, b2, Wfc, bfc):
    f32 = jnp.float32
    i32 = jnp.int32
    src_pad = jnp.concatenate(
        [edge_index[0].astype(i32), jnp.zeros((_E_PAD - _E,), i32)])
    dst_pad = jnp.concatenate(
        [edge_index[1].astype(i32), jnp.full((_E_PAD - _E,), _N, i32)])
    dst2d = dst_pad.reshape(_E_PAD // _EBLK, _EBLK)
    batch2d = batch.astype(i32)[:, None]
    onesrow = jnp.tile(
        (jnp.arange(_CW) == 0).astype(f32)[None, :], (_EBLK, 1))
    z_deg = jnp.zeros((_NPAD, _CW), f32)
    z1 = jnp.zeros((_NPAD, _H1), f32)

    scat = _edge_scatter_kernel(_H1)
    cnt = _deg_kernel()(dst2d, onesrow, z_deg)
    h1 = _k1_call()(x, W1, cnt)
    acc1 = scat(src_pad, dst2d, h1, z1)
    h2a, h2b = _k2_call()(acc1, h1, cnt, b1.reshape(1, _H1), W2)
    acc2a = scat(src_pad, dst2d, h2a, z1)
    acc2b = scat(src_pad, dst2d, h2b, z1)
    out = _k3_call()(acc2a, acc2b, h2a, h2b, cnt, b2.reshape(1, _H2),
                     batch2d, Wfc, bfc.reshape(1, _D_OUT))
    return out


# speedup vs baseline: 1.0033x; 1.0033x over previous
import functools

import jax
import jax.numpy as jnp
from jax import lax
from jax.experimental import pallas as pl
from jax.experimental.pallas import tpu as pltpu
from jax.experimental.pallas import tpu_sc as plsc

_N = 10000
_E = 320000
_D_IN = 128
_H1 = 64
_H2 = 128
_D_OUT = 256
_B = 64

_NC = 2
_NS = 16
_NW = _NC * _NS
_EBLK = 128
_GRP = 4
_E_PER_TILE = 10240
_NBLK = _E_PER_TILE // _EBLK
_NGRP = _NBLK // _GRP
_E_PAD = _E_PER_TILE * _NW
_NPAD = 10048
_RPT = _NPAD // _NS
_CW = 8

_R = 1000
_GRID = _N // _R


def _sc_mesh():
    return plsc.VectorSubcoreMesh(core_axis_name="c", subcore_axis_name="s")


@functools.lru_cache(maxsize=None)
def _deg_kernel():
    @functools.partial(
        pl.kernel,
        out_type=jax.ShapeDtypeStruct((_NC, _NPAD, _CW), jnp.float32),
        mesh=_sc_mesh(),
        compiler_params=pltpu.CompilerParams(use_tc_tiling_on_sc=False),
        scratch_types=[
            pltpu.VMEM((_NBLK, _EBLK), jnp.int32),
            pltpu.VMEM((_EBLK, _CW), jnp.float32),
            pltpu.VMEM_SHARED((_NPAD, _CW), jnp.float32),
        ],
    )
    def k(dst2d_hbm, ones_hbm, zero_hbm, out_hbm, dstbuf, ones_v, degS):
        c = lax.axis_index("c")
        s = lax.axis_index("s")
        w = s * _NC + c
        r0 = s * _RPT
        pltpu.sync_copy(zero_hbm.at[pl.ds(r0, _RPT)],
                        degS.at[pl.ds(r0, _RPT)])
        pltpu.sync_copy(dst2d_hbm.at[pl.ds(w * _NBLK, _NBLK)], dstbuf)
        pltpu.sync_copy(ones_hbm, ones_v)
        plsc.subcore_barrier()

        def blk(j, carry):
            pltpu.sync_copy(ones_v, degS.at[dstbuf.at[j]], add=True)
            return carry

        lax.fori_loop(0, _NBLK, blk, 0)
        plsc.subcore_barrier()
        pltpu.sync_copy(degS.at[pl.ds(r0, _RPT)],
                        out_hbm.at[c, pl.ds(r0, _RPT)])

    return k


@functools.lru_cache(maxsize=None)
def _edge_scatter_kernel(d):
    rows_per_buf = _GRP * _EBLK

    @functools.partial(
        pl.kernel,
        out_type=jax.ShapeDtypeStruct((_NC, _NPAD, d), jnp.float32),
        mesh=_sc_mesh(),
        compiler_params=pltpu.CompilerParams(use_tc_tiling_on_sc=False),
        scratch_types=[
            pltpu.VMEM((_E_PER_TILE,), jnp.int32),
            pltpu.VMEM((_NBLK, _EBLK), jnp.int32),
            pltpu.VMEM((rows_per_buf, d), jnp.float32),
            pltpu.VMEM((rows_per_buf, d), jnp.float32),
            pltpu.VMEM_SHARED((_NPAD, d), jnp.float32),
            pltpu.SemaphoreType.DMA,
            pltpu.SemaphoreType.DMA,
            pltpu.SemaphoreType.DMA,
            pltpu.SemaphoreType.DMA,
        ],
    )
    def k(src_hbm, dst2d_hbm, h_hbm, zero_hbm, out_hbm,
          srcflat, dstbuf, rows0, rows1, accS, gsem0, gsem1, ssem0, ssem1):
        c = lax.axis_index("c")
        s = lax.axis_index("s")
        w = s * _NC + c
        r0 = s * _RPT
        pltpu.sync_copy(zero_hbm.at[pl.ds(r0, _RPT)],
                        accS.at[pl.ds(r0, _RPT)])
        pltpu.sync_copy(src_hbm.at[pl.ds(w * _E_PER_TILE, _E_PER_TILE)],
                        srcflat)
        pltpu.sync_copy(dst2d_hbm.at[pl.ds(w * _NBLK, _NBLK)], dstbuf)
        plsc.subcore_barrier()

        def gather_grp(grp, buf, sem):
            for kk in range(_GRP):
                idx = srcflat.at[pl.ds((grp * _GRP + kk) * _EBLK, _EBLK)]
                pltpu.async_copy(h_hbm.at[idx],
                                 buf.at[pl.ds(kk * _EBLK, _EBLK)], sem)

        def buf_wait(buf, sem):
            pltpu.make_async_copy(
                zero_hbm.at[pl.ds(0, rows_per_buf)], buf, sem).wait()

        def scat_grp(grp, buf, sem):
            for kk in range(_GRP):
                pltpu.async_copy(buf.at[pl.ds(kk * _EBLK, _EBLK)],
                                 accS.at[dstbuf.at[grp * _GRP + kk]], sem,
                                 add=True)

        gather_grp(0, rows0, gsem0)
        gather_grp(1, rows1, gsem1)

        def body(g, carry):
            for kk, (buf, gsem, ssem) in enumerate(
                    ((rows0, gsem0, ssem0), (rows1, gsem1, ssem1))):
                grp = 2 * g + kk
                buf_wait(buf, gsem)
                scat_grp(grp, buf, ssem)
                buf_wait(buf, ssem)
                gather_grp(jnp.minimum(grp + 2, _NGRP - 1), buf, gsem)
            return carry

        lax.fori_loop(0, _NGRP // 2, body, 0)
        buf_wait(rows0, gsem0)
        buf_wait(rows1, gsem1)
        plsc.subcore_barrier()
        pltpu.sync_copy(accS.at[pl.ds(r0, _RPT)],
                        out_hbm.at[c, pl.ds(r0, _RPT)])

    return k


def _dinv(cnt_blk):
    return lax.rsqrt(1.0 + cnt_blk[0][:, 0:1] + cnt_blk[1][:, 0:1])


def _k1_body(x_ref, w_ref, cnt_ref, o_ref):
    g = jnp.dot(x_ref[...], w_ref[...], preferred_element_type=jnp.float32)
    o_ref[...] = g * _dinv(cnt_ref)


def _k2_body(acc_ref, h_ref, cnt_ref, b_ref, w2_ref, oa_ref, ob_ref):
    dinv = _dinv(cnt_ref)
    t = jnp.maximum(dinv * (acc_ref[0] + acc_ref[1] + h_ref[...]) + b_ref[...],
                    0.0)
    h2 = jnp.dot(t, w2_ref[...], preferred_element_type=jnp.float32) * dinv
    oa_ref[...] = h2[:, :_H1]
    ob_ref[...] = h2[:, _H1:]


def _k3_body(acca_ref, accb_ref, ha_ref, hb_ref, cnt_ref, b_ref, batch_ref,
             wfc_ref, bfc_ref, o_ref, pooled, counts):
    i = pl.program_id(0)

    @pl.when(i == 0)
    def _():
        pooled[...] = jnp.zeros_like(pooled)
        counts[...] = jnp.zeros_like(counts)

    dinv = _dinv(cnt_ref)
    pre = jnp.concatenate(
        [acca_ref[0] + acca_ref[1] + ha_ref[...],
         accb_ref[0] + accb_ref[1] + hb_ref[...]], axis=1)
    out2 = jnp.maximum(dinv * pre + b_ref[...], 0.0)
    oh = (batch_ref[...] == lax.broadcasted_iota(jnp.int32, (1, _B), 1)
          ).astype(jnp.float32)
    cdims = (((0,), (0,)), ((), ()))
    pooled[...] += lax.dot_general(oh, out2, cdims,
                                   preferred_element_type=jnp.float32)
    counts[...] += lax.dot_general(oh, jnp.ones((_R, 1), jnp.float32), cdims,
                                   preferred_element_type=jnp.float32)

    @pl.when(i == _GRID - 1)
    def _():
        pm = pooled[...] / jnp.maximum(counts[...], 1.0)
        o_ref[...] = jnp.maximum(
            jnp.dot(pm, wfc_ref[...], preferred_element_type=jnp.float32)
            + bfc_ref[...], 0.0)


def _row_spec(d):
    return pl.BlockSpec((_R, d), lambda i: (i, 0))


def _full_spec(shape):
    return pl.BlockSpec(shape, lambda i: tuple(0 for _ in shape))


def _cnt_spec():
    return pl.BlockSpec((_NC, _R, _CW), lambda i: (0, i, 0))


def _acc_spec(d):
    return pl.BlockSpec((_NC, _R, d), lambda i: (0, i, 0))


@functools.lru_cache(maxsize=None)
def _k1_call():
    return pl.pallas_call(
        _k1_body,
        grid=(_GRID,),
        in_specs=[_row_spec(_D_IN), _full_spec((_D_IN, _H1)), _cnt_spec()],
        out_specs=_row_spec(_H1),
        out_shape=jax.ShapeDtypeStruct((_N, _H1), jnp.float32),
    )


@functools.lru_cache(maxsize=None)
def _k2_call():
    return pl.pallas_call(
        _k2_body,
        grid=(_GRID,),
        in_specs=[_acc_spec(_H1), _row_spec(_H1), _cnt_spec(),
                  _full_spec((1, _H1)), _full_spec((_H1, _H2))],
        out_specs=[_row_spec(_H1), _row_spec(_H1)],
        out_shape=[jax.ShapeDtypeStruct((_N, _H1), jnp.float32),
                   jax.ShapeDtypeStruct((_N, _H1), jnp.float32)],
    )


@functools.lru_cache(maxsize=None)
def _k3_call():
    return pl.pallas_call(
        _k3_body,
        grid=(_GRID,),
        in_specs=[_acc_spec(_H1), _acc_spec(_H1), _row_spec(_H1),
                  _row_spec(_H1), _cnt_spec(),
                  _full_spec((1, _H2)), _row_spec(1),
                  _full_spec((_H2, _D_OUT)), _full_spec((1, _D_OUT))],
        out_specs=_full_spec((_B, _D_OUT)),
        out_shape=jax.ShapeDtypeStruct((_B, _D_OUT), jnp.float32),
        scratch_shapes=[pltpu.VMEM((_B, _H2), jnp.float32),
                        pltpu.VMEM((_B, 1), jnp.float32)],
    )


def kernel(x, edge_index, batch, W1, b1, W2, b2, Wfc, bfc):
    f32 = jnp.float32
    i32 = jnp.int32
    src_pad = jnp.concatenate(
        [edge_index[0].astype(i32), jnp.zeros((_E_PAD - _E,), i32)])
    pad_dst = _N + jnp.arange(_E_PAD - _E, dtype=i32) % (_NPAD - _N)
    dst_pad = jnp.concatenate([edge_index[1].astype(i32), pad_dst])
    dst2d = dst_pad.reshape(_E_PAD // _EBLK, _EBLK)
    batch2d = batch.astype(i32)[:, None]
    onesrow = jnp.tile(
        (jnp.arange(_CW) == 0).astype(f32)[None, :], (_EBLK, 1))
    z_deg = jnp.zeros((_NPAD, _CW), f32)
    z1 = jnp.zeros((_NPAD, _H1), f32)

    scat = _edge_scatter_kernel(_H1)
    cnt = _deg_kernel()(dst2d, onesrow, z_deg)
    h1 = _k1_call()(x, W1, cnt)
    acc1 = scat(src_pad, dst2d, h1, z1)
    h2a, h2b = _k2_call()(acc1, h1, cnt, b1.reshape(1, _H1), W2)
    acc2a = scat(src_pad, dst2d, h2a, z1)
    acc2b = scat(src_pad, dst2d, h2b, z1)
    out = _k3_call()(acc2a, acc2b, h2a, h2b, cnt, b2.reshape(1, _H2),
                     batch2d, Wfc, bfc.reshape(1, _D_OUT))
    return out


# speedup vs baseline: 2.8390x; 2.8296x over previous
import functools

import jax
import jax.numpy as jnp
from jax import lax
from jax.experimental import pallas as pl
from jax.experimental.pallas import tpu as pltpu
from jax.experimental.pallas import tpu_sc as plsc

_N = 10000
_E = 320000
_D_IN = 128
_H1 = 64
_H2 = 128
_D_OUT = 256
_B = 64

_NC = 2
_NS = 16
_NW = _NC * _NS
_EBLK = 128
_GRP = 4
_E_PER_TILE = 10240
_NBLK = _E_PER_TILE // _EBLK
_NGRP = _NBLK // _GRP
_E_PAD = _E_PER_TILE * _NW
_NPAD = 10048
_RPT = _NPAD // _NS
_CW = 8

_R = 1000
_GRID = _N // _R


def _sc_mesh():
    return plsc.VectorSubcoreMesh(core_axis_name="c", subcore_axis_name="s")


@functools.lru_cache(maxsize=None)
def _deg_kernel():
    @functools.partial(
        pl.kernel,
        out_type=jax.ShapeDtypeStruct((_NC, _NPAD, _CW), jnp.float32),
        mesh=_sc_mesh(),
        compiler_params=pltpu.CompilerParams(use_tc_tiling_on_sc=False),
        scratch_types=[
            pltpu.VMEM((_NBLK, _EBLK), jnp.int32),
            pltpu.VMEM((_EBLK, _CW), jnp.float32),
            pltpu.VMEM_SHARED((_NPAD, _CW), jnp.float32),
        ],
    )
    def k(dst2d_hbm, ones_hbm, zero_hbm, out_hbm, dstbuf, ones_v, degS):
        c = lax.axis_index("c")
        s = lax.axis_index("s")
        w = s * _NC + c
        r0 = s * _RPT
        pltpu.sync_copy(zero_hbm.at[pl.ds(r0, _RPT)],
                        degS.at[pl.ds(r0, _RPT)])
        pltpu.sync_copy(dst2d_hbm.at[pl.ds(w * _NBLK, _NBLK)], dstbuf)
        pltpu.sync_copy(ones_hbm, ones_v)
        plsc.subcore_barrier()

        def blk(j, carry):
            pltpu.sync_copy(ones_v, degS.at[dstbuf.at[j]], add=True)
            return carry

        lax.fori_loop(0, _NBLK, blk, 0)
        plsc.subcore_barrier()
        pltpu.sync_copy(degS.at[pl.ds(r0, _RPT)],
                        out_hbm.at[c, pl.ds(r0, _RPT)])

    return k


@functools.lru_cache(maxsize=None)
def _edge_scatter_kernel(d):
    rows_per_buf = _GRP * _EBLK

    @functools.partial(
        pl.kernel,
        out_type=jax.ShapeDtypeStruct((_NC, _NPAD, d), jnp.float32),
        mesh=_sc_mesh(),
        compiler_params=pltpu.CompilerParams(use_tc_tiling_on_sc=False),
        scratch_types=[
            pltpu.VMEM((_E_PER_TILE,), jnp.int32),
            pltpu.VMEM((_NBLK, _EBLK), jnp.int32),
            pltpu.VMEM((rows_per_buf, d), jnp.float32),
            pltpu.VMEM((rows_per_buf, d), jnp.float32),
            pltpu.VMEM_SHARED((_NPAD, d), jnp.float32),
            pltpu.SemaphoreType.DMA,
            pltpu.SemaphoreType.DMA,
            pltpu.SemaphoreType.DMA,
            pltpu.SemaphoreType.DMA,
        ],
    )
    def k(src_hbm, dst2d_hbm, h_hbm, zero_hbm, out_hbm,
          srcflat, dstbuf, rows0, rows1, accS, gsem0, gsem1, ssem0, ssem1):
        c = lax.axis_index("c")
        s = lax.axis_index("s")
        w = s * _NC + c
        r0 = s * _RPT
        pltpu.sync_copy(zero_hbm.at[pl.ds(r0, _RPT)],
                        accS.at[pl.ds(r0, _RPT)])
        pltpu.sync_copy(src_hbm.at[pl.ds(w * _E_PER_TILE, _E_PER_TILE)],
                        srcflat)
        pltpu.sync_copy(dst2d_hbm.at[pl.ds(w * _NBLK, _NBLK)], dstbuf)
        plsc.subcore_barrier()

        def gather_grp(grp, buf, sem):
            for kk in range(_GRP):
                idx = srcflat.at[pl.ds((grp * _GRP + kk) * _EBLK, _EBLK)]
                pltpu.async_copy(h_hbm.at[idx],
                                 buf.at[pl.ds(kk * _EBLK, _EBLK)], sem)

        def buf_wait(buf, sem):
            pltpu.make_async_copy(
                zero_hbm.at[pl.ds(0, rows_per_buf)], buf, sem).wait()

        def scat_grp(grp, buf, sem):
            for kk in range(_GRP):
                pltpu.async_copy(buf.at[pl.ds(kk * _EBLK, _EBLK)],
                                 accS.at[dstbuf.at[grp * _GRP + kk]], sem,
                                 add=True)

        gather_grp(0, rows0, gsem0)
        gather_grp(1, rows1, gsem1)

        def body(g, carry):
            for kk, (buf, gsem, ssem) in enumerate(
                    ((rows0, gsem0, ssem0), (rows1, gsem1, ssem1))):
                grp = 2 * g + kk
                buf_wait(buf, gsem)
                scat_grp(grp, buf, ssem)
                buf_wait(buf, ssem)
                gather_grp(jnp.minimum(grp + 2, _NGRP - 1), buf, gsem)
            return carry

        lax.fori_loop(0, _NGRP // 2, body, 0)
        buf_wait(rows0, gsem0)
        buf_wait(rows1, gsem1)
        plsc.subcore_barrier()
        pltpu.sync_copy(accS.at[pl.ds(r0, _RPT)],
                        out_hbm.at[c, pl.ds(r0, _RPT)])

    return k


def _dinv(cnt_blk):
    return lax.rsqrt(1.0 + cnt_blk[0][:, 0:1] + cnt_blk[1][:, 0:1])


def _k1_body(x_ref, w_ref, cnt_ref, o_ref):
    g = jnp.dot(x_ref[...], w_ref[...], preferred_element_type=jnp.float32)
    o_ref[...] = g * _dinv(cnt_ref)


def _k2_body(acc_ref, h_ref, cnt_ref, b_ref, w2_ref, oa_ref, ob_ref):
    dinv = _dinv(cnt_ref)
    t = jnp.maximum(dinv * (acc_ref[0] + acc_ref[1] + h_ref[...]) + b_ref[...],
                    0.0)
    h2 = jnp.dot(t, w2_ref[...], preferred_element_type=jnp.float32) * dinv
    oa_ref[...] = h2[:, :_H1]
    ob_ref[...] = h2[:, _H1:]


def _k3_body(acca_ref, accb_ref, ha_ref, hb_ref, cnt_ref, b_ref, batch_ref,
             wfc_ref, bfc_ref, o_ref, pooled, counts):
    i = pl.program_id(0)

    @pl.when(i == 0)
    def _():
        pooled[...] = jnp.zeros_like(pooled)
        counts[...] = jnp.zeros_like(counts)

    dinv = _dinv(cnt_ref)
    pre = jnp.concatenate(
        [acca_ref[0] + acca_ref[1] + ha_ref[...],
         accb_ref[0] + accb_ref[1] + hb_ref[...]], axis=1)
    out2 = jnp.maximum(dinv * pre + b_ref[...], 0.0)
    oh = (batch_ref[...] == lax.broadcasted_iota(jnp.int32, (1, _B), 1)
          ).astype(jnp.float32)
    cdims = (((0,), (0,)), ((), ()))
    pooled[...] += lax.dot_general(oh, out2, cdims,
                                   preferred_element_type=jnp.float32)
    counts[...] += lax.dot_general(oh, jnp.ones((_R, 1), jnp.float32), cdims,
                                   preferred_element_type=jnp.float32)

    @pl.when(i == _GRID - 1)
    def _():
        pm = pooled[...] / jnp.maximum(counts[...], 1.0)
        o_ref[...] = jnp.maximum(
            jnp.dot(pm, wfc_ref[...], preferred_element_type=jnp.float32)
            + bfc_ref[...], 0.0)


def _row_spec(d):
    return pl.BlockSpec((_R, d), lambda i: (i, 0))


def _full_spec(shape):
    return pl.BlockSpec(shape, lambda i: tuple(0 for _ in shape))


def _cnt_spec():
    return pl.BlockSpec((_NC, _R, _CW), lambda i: (0, i, 0))


def _acc_spec(d):
    return pl.BlockSpec((_NC, _R, d), lambda i: (0, i, 0))


@functools.lru_cache(maxsize=None)
def _k1_call():
    return pl.pallas_call(
        _k1_body,
        grid=(_GRID,),
        in_specs=[_row_spec(_D_IN), _full_spec((_D_IN, _H1)), _cnt_spec()],
        out_specs=_row_spec(_H1),
        out_shape=jax.ShapeDtypeStruct((_N, _H1), jnp.float32),
    )


@functools.lru_cache(maxsize=None)
def _k2_call():
    return pl.pallas_call(
        _k2_body,
        grid=(_GRID,),
        in_specs=[_acc_spec(_H1), _row_spec(_H1), _cnt_spec(),
                  _full_spec((1, _H1)), _full_spec((_H1, _H2))],
        out_specs=[_row_spec(_H1), _row_spec(_H1)],
        out_shape=[jax.ShapeDtypeStruct((_N, _H1), jnp.float32),
                   jax.ShapeDtypeStruct((_N, _H1), jnp.float32)],
    )


@functools.lru_cache(maxsize=None)
def _k3_call():
    return pl.pallas_call(
        _k3_body,
        grid=(_GRID,),
        in_specs=[_acc_spec(_H1), _acc_spec(_H1), _row_spec(_H1),
                  _row_spec(_H1), _cnt_spec(),
                  _full_spec((1, _H2)), _row_spec(1),
                  _full_spec((_H2, _D_OUT)), _full_spec((1, _D_OUT))],
        out_specs=_full_spec((_B, _D_OUT)),
        out_shape=jax.ShapeDtypeStruct((_B, _D_OUT), jnp.float32),
        scratch_shapes=[pltpu.VMEM((_B, _H2), jnp.float32),
                        pltpu.VMEM((_B, 1), jnp.float32)],
    )


def kernel(x, edge_index, batch, W1, b1, W2, b2, Wfc, bfc):
    f32 = jnp.float32
    i32 = jnp.int32
    pad_src = jnp.arange(_E_PAD - _E, dtype=i32) % _N
    src_pad = jnp.concatenate([edge_index[0].astype(i32), pad_src])
    pad_dst = _N + jnp.arange(_E_PAD - _E, dtype=i32) % (_NPAD - _N)
    dst_pad = jnp.concatenate([edge_index[1].astype(i32), pad_dst])
    dst2d = dst_pad.reshape(_E_PAD // _EBLK, _EBLK)
    batch2d = batch.astype(i32)[:, None]
    onesrow = jnp.tile(
        (jnp.arange(_CW) == 0).astype(f32)[None, :], (_EBLK, 1))
    z_deg = jnp.zeros((_NPAD, _CW), f32)
    z1 = jnp.zeros((_NPAD, _H1), f32)

    scat = _edge_scatter_kernel(_H1)
    cnt = _deg_kernel()(dst2d, onesrow, z_deg)
    h1 = _k1_call()(x, W1, cnt)
    acc1 = scat(src_pad, dst2d, h1, z1)
    h2a, h2b = _k2_call()(acc1, h1, cnt, b1.reshape(1, _H1), W2)
    acc2a = scat(src_pad, dst2d, h2a, z1)
    acc2b = scat(src_pad, dst2d, h2b, z1)
    out = _k3_call()(acc2a, acc2b, h2a, h2b, cnt, b2.reshape(1, _H2),
                     batch2d, Wfc, bfc.reshape(1, _D_OUT))
    return out


# speedup vs baseline: 2.8395x; 1.0002x over previous
import functools

import jax
import jax.numpy as jnp
from jax import lax
from jax.experimental import pallas as pl
from jax.experimental.pallas import tpu as pltpu
from jax.experimental.pallas import tpu_sc as plsc

_N = 10000
_E = 320000
_D_IN = 128
_H1 = 64
_H2 = 128
_D_OUT = 256
_B = 64

_NC = 2
_NS = 16
_NW = _NC * _NS
_EBLK = 128
_GRP = 4
_NBUF = 2
_E_PER_TILE = 10240
_NBLK = _E_PER_TILE // _EBLK
_NGRP = _NBLK // _GRP
_E_PAD = _E_PER_TILE * _NW
_NPAD = 10048
_RPT = _NPAD // _NS
_CW = 8

_R = 1000
_GRID = _N // _R


def _sc_mesh():
    return plsc.VectorSubcoreMesh(core_axis_name="c", subcore_axis_name="s")


@functools.lru_cache(maxsize=None)
def _deg_kernel():
    @functools.partial(
        pl.kernel,
        out_type=jax.ShapeDtypeStruct((_NC, _NPAD, _CW), jnp.float32),
        mesh=_sc_mesh(),
        compiler_params=pltpu.CompilerParams(use_tc_tiling_on_sc=False),
        scratch_types=[
            pltpu.VMEM((_NBLK, _EBLK), jnp.int32),
            pltpu.VMEM((_EBLK, _CW), jnp.float32),
            pltpu.VMEM_SHARED((_NPAD, _CW), jnp.float32),
        ],
    )
    def k(dst2d_hbm, ones_hbm, zero_hbm, out_hbm, dstbuf, ones_v, degS):
        c = lax.axis_index("c")
        s = lax.axis_index("s")
        w = s * _NC + c
        r0 = s * _RPT
        pltpu.sync_copy(zero_hbm.at[pl.ds(r0, _RPT)],
                        degS.at[pl.ds(r0, _RPT)])
        pltpu.sync_copy(dst2d_hbm.at[pl.ds(w * _NBLK, _NBLK)], dstbuf)
        pltpu.sync_copy(ones_hbm, ones_v)
        plsc.subcore_barrier()

        def blk(j, carry):
            pltpu.sync_copy(ones_v, degS.at[dstbuf.at[j]], add=True)
            return carry

        lax.fori_loop(0, _NBLK, blk, 0)
        plsc.subcore_barrier()
        pltpu.sync_copy(degS.at[pl.ds(r0, _RPT)],
                        out_hbm.at[c, pl.ds(r0, _RPT)])

    return k


@functools.lru_cache(maxsize=None)
def _edge_scatter_kernel(d):
    rows_per_buf = _GRP * _EBLK
    n_loop = _NGRP // _NBUF
    n_tail = _NGRP - n_loop * _NBUF

    @functools.partial(
        pl.kernel,
        out_type=jax.ShapeDtypeStruct((_NC, _NPAD, d), jnp.float32),
        mesh=_sc_mesh(),
        compiler_params=pltpu.CompilerParams(use_tc_tiling_on_sc=False),
        scratch_types=[
            pltpu.VMEM((_E_PER_TILE,), jnp.int32),
            pltpu.VMEM((_NBLK, _EBLK), jnp.int32),
            pltpu.VMEM((rows_per_buf, d), jnp.float32),
            pltpu.VMEM((rows_per_buf, d), jnp.float32),
            pltpu.VMEM_SHARED((_NPAD, d), jnp.float32),
            pltpu.SemaphoreType.DMA,
            pltpu.SemaphoreType.DMA,
            pltpu.SemaphoreType.DMA,
            pltpu.SemaphoreType.DMA,
        ],
    )
    def k(src_hbm, dst2d_hbm, h_hbm, zero_hbm, out_hbm,
          srcflat, dstbuf, rows0, rows1, accS, gsem0, gsem1, ssem0, ssem1):
        rows = (rows0, rows1)
        gsems = (gsem0, gsem1)
        ssems = (ssem0, ssem1)
        c = lax.axis_index("c")
        s = lax.axis_index("s")
        w = s * _NC + c
        r0 = s * _RPT
        pltpu.sync_copy(zero_hbm.at[pl.ds(r0, _RPT)],
                        accS.at[pl.ds(r0, _RPT)])
        pltpu.sync_copy(src_hbm.at[pl.ds(w * _E_PER_TILE, _E_PER_TILE)],
                        srcflat)
        pltpu.sync_copy(dst2d_hbm.at[pl.ds(w * _NBLK, _NBLK)], dstbuf)
        plsc.subcore_barrier()

        def gather_grp(grp, b):
            for kk in range(_GRP):
                idx = srcflat.at[pl.ds((grp * _GRP + kk) * _EBLK, _EBLK)]
                pltpu.async_copy(h_hbm.at[idx],
                                 rows[b].at[pl.ds(kk * _EBLK, _EBLK)],
                                 gsems[b])

        def buf_wait(b, sem):
            pltpu.make_async_copy(
                zero_hbm.at[pl.ds(0, rows_per_buf)], rows[b], sem).wait()

        def scat_grp(grp, b):
            for kk in range(_GRP):
                pltpu.async_copy(rows[b].at[pl.ds(kk * _EBLK, _EBLK)],
                                 accS.at[dstbuf.at[grp * _GRP + kk]],
                                 ssems[b], add=True)

        for b in range(_NBUF):
            gather_grp(b, b)

        def slot(grp, b, prefetch):
            buf_wait(b, gsems[b])
            scat_grp(grp, b)
            buf_wait(b, ssems[b])
            if prefetch:
                gather_grp(jnp.minimum(grp + _NBUF, _NGRP - 1), b)

        def body(g, carry):
            for b in range(_NBUF):
                slot(_NBUF * g + b, b, True)
            return carry

        lax.fori_loop(0, n_loop, body, 0)
        for b in range(n_tail):
            slot(n_loop * _NBUF + b, b, False)
        for b in range(n_tail, _NBUF):
            buf_wait(b, gsems[b])
        plsc.subcore_barrier()
        pltpu.sync_copy(accS.at[pl.ds(r0, _RPT)],
                        out_hbm.at[c, pl.ds(r0, _RPT)])

    return k


def _dinv(cnt_blk):
    return lax.rsqrt(1.0 + cnt_blk[0][:, 0:1] + cnt_blk[1][:, 0:1])


def _k1_body(x_ref, w_ref, cnt_ref, o_ref):
    g = jnp.dot(x_ref[...], w_ref[...], preferred_element_type=jnp.float32)
    o_ref[...] = g * _dinv(cnt_ref)


def _k2_body(acc_ref, h_ref, cnt_ref, b_ref, w2_ref, oa_ref, ob_ref):
    dinv = _dinv(cnt_ref)
    t = jnp.maximum(dinv * (acc_ref[0] + acc_ref[1] + h_ref[...]) + b_ref[...],
                    0.0)
    h2 = jnp.dot(t, w2_ref[...], preferred_element_type=jnp.float32) * dinv
    oa_ref[...] = h2[:, :_H1]
    ob_ref[...] = h2[:, _H1:]


def _k3_body(acca_ref, accb_ref, ha_ref, hb_ref, cnt_ref, b_ref, batch_ref,
             wfc_ref, bfc_ref, o_ref, pooled, counts):
    i = pl.program_id(0)

    @pl.when(i == 0)
    def _():
        pooled[...] = jnp.zeros_like(pooled)
        counts[...] = jnp.zeros_like(counts)

    dinv = _dinv(cnt_ref)
    pre = jnp.concatenate(
        [acca_ref[0] + acca_ref[1] + ha_ref[...],
         accb_ref[0] + accb_ref[1] + hb_ref[...]], axis=1)
    out2 = jnp.maximum(dinv * pre + b_ref[...], 0.0)
    oh = (batch_ref[...] == lax.broadcasted_iota(jnp.int32, (1, _B), 1)
          ).astype(jnp.float32)
    cdims = (((0,), (0,)), ((), ()))
    pooled[...] += lax.dot_general(oh, out2, cdims,
                                   preferred_element_type=jnp.float32)
    counts[...] += lax.dot_general(oh, jnp.ones((_R, 1), jnp.float32), cdims,
                                   preferred_element_type=jnp.float32)

    @pl.when(i == _GRID - 1)
    def _():
        pm = pooled[...] / jnp.maximum(counts[...], 1.0)
        o_ref[...] = jnp.maximum(
            jnp.dot(pm, wfc_ref[...], preferred_element_type=jnp.float32)
            + bfc_ref[...], 0.0)


def _row_spec(d):
    return pl.BlockSpec((_R, d), lambda i: (i, 0))


def _full_spec(shape):
    return pl.BlockSpec(shape, lambda i: tuple(0 for _ in shape))


def _cnt_spec():
    return pl.BlockSpec((_NC, _R, _CW), lambda i: (0, i, 0))


def _acc_spec(d):
    return pl.BlockSpec((_NC, _R, d), lambda i: (0, i, 0))


@functools.lru_cache(maxsize=None)
def _k1_call():
    return pl.pallas_call(
        _k1_body,
        grid=(_GRID,),
        in_specs=[_row_spec(_D_IN), _full_spec((_D_IN, _H1)), _cnt_spec()],
        out_specs=_row_spec(_H1),
        out_shape=jax.ShapeDtypeStruct((_N, _H1), jnp.float32),
    )


@functools.lru_cache(maxsize=None)
def _k2_call():
    return pl.pallas_call(
        _k2_body,
        grid=(_GRID,),
        in_specs=[_acc_spec(_H1), _row_spec(_H1), _cnt_spec(),
                  _full_spec((1, _H1)), _full_spec((_H1, _H2))],
        out_specs=[_row_spec(_H1), _row_spec(_H1)],
        out_shape=[jax.ShapeDtypeStruct((_N, _H1), jnp.float32),
                   jax.ShapeDtypeStruct((_N, _H1), jnp.float32)],
    )


@functools.lru_cache(maxsize=None)
def _k3_call():
    return pl.pallas_call(
        _k3_body,
        grid=(_GRID,),
        in_specs=[_acc_spec(_H1), _acc_spec(_H1), _row_spec(_H1),
                  _row_spec(_H1), _cnt_spec(),
                  _full_spec((1, _H2)), _row_spec(1),
                  _full_spec((_H2, _D_OUT)), _full_spec((1, _D_OUT))],
        out_specs=_full_spec((_B, _D_OUT)),
        out_shape=jax.ShapeDtypeStruct((_B, _D_OUT), jnp.float32),
        scratch_shapes=[pltpu.VMEM((_B, _H2), jnp.float32),
                        pltpu.VMEM((_B, 1), jnp.float32)],
    )


def kernel(x, edge_index, batch, W1, b1, W2, b2, Wfc, bfc):
    f32 = jnp.float32
    i32 = jnp.int32
    pad_src = jnp.arange(_E_PAD - _E, dtype=i32) % _N
    src_pad = jnp.concatenate([edge_index[0].astype(i32), pad_src])
    pad_dst = _N + jnp.arange(_E_PAD - _E, dtype=i32) % (_NPAD - _N)
    dst_pad = jnp.concatenate([edge_index[1].astype(i32), pad_dst])
    dst2d = dst_pad.reshape(_E_PAD // _EBLK, _EBLK)
    batch2d = batch.astype(i32)[:, None]
    onesrow = jnp.tile(
        (jnp.arange(_CW) == 0).astype(f32)[None, :], (_EBLK, 1))
    z_deg = jnp.zeros((_NPAD, _CW), f32)
    z1 = jnp.zeros((_NPAD, _H1), f32)

    scat = _edge_scatter_kernel(_H1)
    cnt = _deg_kernel()(dst2d, onesrow, z_deg)
    h1 = _k1_call()(x, W1, cnt)
    acc1 = scat(src_pad, dst2d, h1, z1)
    h2a, h2b = _k2_call()(acc1, h1, cnt, b1.reshape(1, _H1), W2)
    acc2a = scat(src_pad, dst2d, h2a, z1)
    acc2b = scat(src_pad, dst2d, h2b, z1)
    out = _k3_call()(acc2a, acc2b, h2a, h2b, cnt, b2.reshape(1, _H2),
                     batch2d, Wfc, bfc.reshape(1, _D_OUT))
    return out


# speedup vs baseline: 3.6496x; 1.2853x over previous
import functools

import jax
import jax.numpy as jnp
from jax import lax
from jax.experimental import pallas as pl
from jax.experimental.pallas import tpu as pltpu
from jax.experimental.pallas import tpu_sc as plsc

_N = 10000
_E = 320000
_D_IN = 128
_H1 = 64
_H2 = 128
_D_OUT = 256
_B = 64

_NC = 2
_NS = 16
_NW = _NC * _NS
_EBLK = 128
_GRP = 4
_NBUF = 2
_E_PER_TILE = 10240
_NBLK = _E_PER_TILE // _EBLK
_NGRP = _NBLK // _GRP
_E_PAD = _E_PER_TILE * _NW
_NPAD = 10048
_RPT = _NPAD // _NS
_CW = 8

_R = 1000
_GRID = _N // _R


def _sc_mesh():
    return plsc.VectorSubcoreMesh(core_axis_name="c", subcore_axis_name="s")


@functools.lru_cache(maxsize=None)
def _deg_kernel():
    @functools.partial(
        pl.kernel,
        out_type=jax.ShapeDtypeStruct((_NC, _NPAD, _CW), jnp.float32),
        mesh=_sc_mesh(),
        compiler_params=pltpu.CompilerParams(use_tc_tiling_on_sc=False),
        scratch_types=[
            pltpu.VMEM((_NBLK, _EBLK), jnp.int32),
            pltpu.VMEM((_EBLK, _CW), jnp.float32),
            pltpu.VMEM_SHARED((_NPAD, _CW), jnp.float32),
        ],
    )
    def k(dst2d_hbm, ones_hbm, zero_hbm, out_hbm, dstbuf, ones_v, degS):
        c = lax.axis_index("c")
        s = lax.axis_index("s")
        w = s * _NC + c
        r0 = s * _RPT
        pltpu.sync_copy(zero_hbm.at[pl.ds(r0, _RPT)],
                        degS.at[pl.ds(r0, _RPT)])
        pltpu.sync_copy(dst2d_hbm.at[pl.ds(w * _NBLK, _NBLK)], dstbuf)
        pltpu.sync_copy(ones_hbm, ones_v)
        plsc.subcore_barrier()

        def blk(j, carry):
            pltpu.sync_copy(ones_v, degS.at[dstbuf.at[j]], add=True)
            return carry

        lax.fori_loop(0, _NBLK, blk, 0)
        plsc.subcore_barrier()
        pltpu.sync_copy(degS.at[pl.ds(r0, _RPT)],
                        out_hbm.at[c, pl.ds(r0, _RPT)])

    return k


@functools.lru_cache(maxsize=None)
def _edge_scatter_kernel(d):
    rows_per_buf = _GRP * _EBLK
    n_loop = _NGRP // _NBUF
    n_tail = _NGRP - n_loop * _NBUF

    @functools.partial(
        pl.kernel,
        out_type=jax.ShapeDtypeStruct((_NC, _NPAD, d), jnp.float32),
        mesh=_sc_mesh(),
        compiler_params=pltpu.CompilerParams(use_tc_tiling_on_sc=False),
        scratch_types=[
            pltpu.VMEM((_E_PER_TILE,), jnp.int32),
            pltpu.VMEM((_NBLK, _EBLK), jnp.int32),
            pltpu.VMEM((rows_per_buf, d), jnp.float32),
            pltpu.VMEM((rows_per_buf, d), jnp.float32),
            pltpu.VMEM_SHARED((_NPAD, d), jnp.float32),
            pltpu.SemaphoreType.DMA,
            pltpu.SemaphoreType.DMA,
            pltpu.SemaphoreType.DMA,
            pltpu.SemaphoreType.DMA,
        ],
    )
    def k(src_hbm, dst2d_hbm, h_hbm, zero_hbm, out_hbm,
          srcflat, dstbuf, rows0, rows1, accS, gsem0, gsem1, ssem0, ssem1):
        rows = (rows0, rows1)
        gsems = (gsem0, gsem1)
        ssems = (ssem0, ssem1)
        c = lax.axis_index("c")
        s = lax.axis_index("s")
        w = s * _NC + c
        r0 = s * _RPT
        pltpu.sync_copy(zero_hbm.at[pl.ds(r0, _RPT)],
                        accS.at[pl.ds(r0, _RPT)])
        pltpu.sync_copy(src_hbm.at[pl.ds(w * _E_PER_TILE, _E_PER_TILE)],
                        srcflat)
        pltpu.sync_copy(dst2d_hbm.at[pl.ds(w * _NBLK, _NBLK)], dstbuf)
        plsc.subcore_barrier()

        def gather_grp(grp, b):
            for kk in range(_GRP):
                idx = srcflat.at[pl.ds((grp * _GRP + kk) * _EBLK, _EBLK)]
                pltpu.async_copy(h_hbm.at[idx],
                                 rows[b].at[pl.ds(kk * _EBLK, _EBLK)],
                                 gsems[b])

        def buf_wait(b, sem):
            pltpu.make_async_copy(
                zero_hbm.at[pl.ds(0, rows_per_buf)], rows[b], sem).wait()

        def scat_grp(grp, b):
            for kk in range(_GRP):
                pltpu.async_copy(rows[b].at[pl.ds(kk * _EBLK, _EBLK)],
                                 accS.at[dstbuf.at[grp * _GRP + kk]],
                                 ssems[b], add=True)

        for b in range(_NBUF):
            gather_grp(b, b)

        def slot(grp, b, prefetch):
            buf_wait(b, gsems[b])
            scat_grp(grp, b)
            buf_wait(b, ssems[b])
            if prefetch:
                gather_grp(jnp.minimum(grp + _NBUF, _NGRP - 1), b)

        def body(g, carry):
            for b in range(_NBUF):
                slot(_NBUF * g + b, b, True)
            return carry

        lax.fori_loop(0, n_loop, body, 0)
        for b in range(n_tail):
            slot(n_loop * _NBUF + b, b, False)
        for b in range(n_tail, _NBUF):
            buf_wait(b, gsems[b])
        plsc.subcore_barrier()
        pltpu.sync_copy(accS.at[pl.ds(r0, _RPT)],
                        out_hbm.at[c, pl.ds(r0, _RPT)])

    return k


def _dinv(cnt_blk):
    return lax.rsqrt(1.0 + cnt_blk[0][:, 0:1] + cnt_blk[1][:, 0:1])


def _k1_body(x_ref, w_ref, cnt_ref, o_ref):
    g = jnp.dot(x_ref[...], w_ref[...], preferred_element_type=jnp.float32)
    o_ref[...] = g * _dinv(cnt_ref)


def _k2_body(acc_ref, h_ref, cnt_ref, b_ref, o_ref):
    dinv = _dinv(cnt_ref)
    t = jnp.maximum(dinv * (acc_ref[0] + acc_ref[1] + h_ref[...]) + b_ref[...],
                    0.0)
    o_ref[...] = t * dinv


def _k3_body(acc_ref, t_ref, cnt_ref, w2_ref, b_ref, batch_ref,
             wfc_ref, bfc_ref, o_ref, pooled, counts):
    i = pl.program_id(0)

    @pl.when(i == 0)
    def _():
        pooled[...] = jnp.zeros_like(pooled)
        counts[...] = jnp.zeros_like(counts)

    dinv = _dinv(cnt_ref)
    u = dinv * (acc_ref[0] + acc_ref[1] + t_ref[...])
    out2 = jnp.maximum(
        jnp.dot(u, w2_ref[...], preferred_element_type=jnp.float32)
        + b_ref[...], 0.0)
    oh = (batch_ref[...] == lax.broadcasted_iota(jnp.int32, (1, _B), 1)
          ).astype(jnp.float32)
    cdims = (((0,), (0,)), ((), ()))
    pooled[...] += lax.dot_general(oh, out2, cdims,
                                   preferred_element_type=jnp.float32)
    counts[...] += lax.dot_general(oh, jnp.ones((_R, 1), jnp.float32), cdims,
                                   preferred_element_type=jnp.float32)

    @pl.when(i == _GRID - 1)
    def _():
        pm = pooled[...] / jnp.maximum(counts[...], 1.0)
        o_ref[...] = jnp.maximum(
            jnp.dot(pm, wfc_ref[...], preferred_element_type=jnp.float32)
            + bfc_ref[...], 0.0)


def _row_spec(d):
    return pl.BlockSpec((_R, d), lambda i: (i, 0))


def _full_spec(shape):
    return pl.BlockSpec(shape, lambda i: tuple(0 for _ in shape))


def _cnt_spec():
    return pl.BlockSpec((_NC, _R, _CW), lambda i: (0, i, 0))


def _acc_spec(d):
    return pl.BlockSpec((_NC, _R, d), lambda i: (0, i, 0))


@functools.lru_cache(maxsize=None)
def _k1_call():
    return pl.pallas_call(
        _k1_body,
        grid=(_GRID,),
        in_specs=[_row_spec(_D_IN), _full_spec((_D_IN, _H1)), _cnt_spec()],
        out_specs=_row_spec(_H1),
        out_shape=jax.ShapeDtypeStruct((_N, _H1), jnp.float32),
    )


@functools.lru_cache(maxsize=None)
def _k2_call():
    return pl.pallas_call(
        _k2_body,
        grid=(_GRID,),
        in_specs=[_acc_spec(_H1), _row_spec(_H1), _cnt_spec(),
                  _full_spec((1, _H1))],
        out_specs=_row_spec(_H1),
        out_shape=jax.ShapeDtypeStruct((_N, _H1), jnp.float32),
    )


@functools.lru_cache(maxsize=None)
def _k3_call():
    return pl.pallas_call(
        _k3_body,
        grid=(_GRID,),
        in_specs=[_acc_spec(_H1), _row_spec(_H1), _cnt_spec(),
                  _full_spec((_H1, _H2)), _full_spec((1, _H2)), _row_spec(1),
                  _full_spec((_H2, _D_OUT)), _full_spec((1, _D_OUT))],
        out_specs=_full_spec((_B, _D_OUT)),
        out_shape=jax.ShapeDtypeStruct((_B, _D_OUT), jnp.float32),
        scratch_shapes=[pltpu.VMEM((_B, _H2), jnp.float32),
                        pltpu.VMEM((_B, 1), jnp.float32)],
    )


def kernel(x, edge_index, batch, W1, b1, W2, b2, Wfc, bfc):
    f32 = jnp.float32
    i32 = jnp.int32
    pad_src = jnp.arange(_E_PAD - _E, dtype=i32) % _N
    src_pad = jnp.concatenate([edge_index[0].astype(i32), pad_src])
    pad_dst = _N + jnp.arange(_E_PAD - _E, dtype=i32) % (_NPAD - _N)
    dst_pad = jnp.concatenate([edge_index[1].astype(i32), pad_dst])
    dst2d = dst_pad.reshape(_E_PAD // _EBLK, _EBLK)
    batch2d = batch.astype(i32)[:, None]
    onesrow = jnp.tile(
        (jnp.arange(_CW) == 0).astype(f32)[None, :], (_EBLK, 1))
    z_deg = jnp.zeros((_NPAD, _CW), f32)
    z1 = jnp.zeros((_NPAD, _H1), f32)

    scat = _edge_scatter_kernel(_H1)
    cnt = _deg_kernel()(dst2d, onesrow, z_deg)
    h1 = _k1_call()(x, W1, cnt)
    acc1 = scat(src_pad, dst2d, h1, z1)
    t = _k2_call()(acc1, h1, cnt, b1.reshape(1, _H1))
    acc2 = scat(src_pad, dst2d, t, z1)
    out = _k3_call()(acc2, t, cnt, W2, b2.reshape(1, _H2),
                     batch2d, Wfc, bfc.reshape(1, _D_OUT))
    return out


# speedup vs baseline: 3.7006x; 1.0140x over previous
import functools

import jax
import jax.numpy as jnp
from jax import lax
from jax.experimental import pallas as pl
from jax.experimental.pallas import tpu as pltpu
from jax.experimental.pallas import tpu_sc as plsc

_N = 10000
_E = 320000
_D_IN = 128
_H1 = 64
_H2 = 128
_D_OUT = 256
_B = 64

_NC = 2
_NS = 16
_NW = _NC * _NS
_EBLK = 128
_GRP = 4
_NBUF = 2
_E_PER_TILE = 10240
_NBLK = _E_PER_TILE // _EBLK
_NGRP = _NBLK // _GRP
_E_PAD = _E_PER_TILE * _NW
_NPAD = 10048
_RPT = _NPAD // _NS
_CW = 8

_R = 1000
_GRID = _N // _R


def _sc_mesh():
    return plsc.VectorSubcoreMesh(core_axis_name="c", subcore_axis_name="s")


@functools.lru_cache(maxsize=None)
def _deg_kernel():
    @functools.partial(
        pl.kernel,
        out_type=jax.ShapeDtypeStruct((_NC, _NPAD, _CW), jnp.float32),
        mesh=_sc_mesh(),
        compiler_params=pltpu.CompilerParams(use_tc_tiling_on_sc=False),
        scratch_types=[
            pltpu.VMEM((_NBLK, _EBLK), jnp.int32),
            pltpu.VMEM((_EBLK, _CW), jnp.float32),
            pltpu.VMEM_SHARED((_NPAD, _CW), jnp.float32),
        ],
    )
    def k(dst2d_hbm, ones_hbm, zero_hbm, out_hbm, dstbuf, ones_v, degS):
        c = lax.axis_index("c")
        s = lax.axis_index("s")
        w = s * _NC + c
        r0 = s * _RPT
        pltpu.sync_copy(zero_hbm.at[pl.ds(r0, _RPT)],
                        degS.at[pl.ds(r0, _RPT)])
        pltpu.sync_copy(dst2d_hbm.at[pl.ds(w * _NBLK, _NBLK)], dstbuf)
        pltpu.sync_copy(ones_hbm, ones_v)
        plsc.subcore_barrier()

        def blk(j, carry):
            pltpu.sync_copy(ones_v, degS.at[dstbuf.at[j]], add=True)
            return carry

        lax.fori_loop(0, _NBLK, blk, 0)
        plsc.subcore_barrier()
        pltpu.sync_copy(degS.at[pl.ds(r0, _RPT)],
                        out_hbm.at[c, pl.ds(r0, _RPT)])

    return k


@functools.lru_cache(maxsize=None)
def _edge_scatter_kernel(d):
    rows_per_buf = _GRP * _EBLK
    n_loop = _NGRP // _NBUF
    n_tail = _NGRP - n_loop * _NBUF

    @functools.partial(
        pl.kernel,
        out_type=jax.ShapeDtypeStruct((_NC, _NPAD, d), jnp.float32),
        mesh=_sc_mesh(),
        compiler_params=pltpu.CompilerParams(use_tc_tiling_on_sc=False),
        scratch_types=[
            pltpu.VMEM((_E_PER_TILE,), jnp.int32),
            pltpu.VMEM((_NBLK, _EBLK), jnp.int32),
            pltpu.VMEM((rows_per_buf, d), jnp.float32),
            pltpu.VMEM((rows_per_buf, d), jnp.float32),
            pltpu.VMEM_SHARED((_NPAD, d), jnp.float32),
            pltpu.SemaphoreType.DMA,
            pltpu.SemaphoreType.DMA,
            pltpu.SemaphoreType.DMA,
            pltpu.SemaphoreType.DMA,
        ],
    )
    def k(src_hbm, dst2d_hbm, h_hbm, zero_hbm, out_hbm,
          srcflat, dstbuf, rows0, rows1, accS, gsem0, gsem1, ssem0, ssem1):
        rows = (rows0, rows1)
        gsems = (gsem0, gsem1)
        ssems = (ssem0, ssem1)
        c = lax.axis_index("c")
        s = lax.axis_index("s")
        w = s * _NC + c
        r0 = s * _RPT
        pltpu.async_copy(zero_hbm.at[pl.ds(r0, _RPT)],
                         accS.at[pl.ds(r0, _RPT)], ssem0)
        pltpu.async_copy(src_hbm.at[pl.ds(w * _E_PER_TILE, _E_PER_TILE)],
                         srcflat, ssem0)
        pltpu.async_copy(dst2d_hbm.at[pl.ds(w * _NBLK, _NBLK)], dstbuf, ssem0)
        pltpu.make_async_copy(zero_hbm.at[pl.ds(r0, _RPT)],
                              accS.at[pl.ds(r0, _RPT)], ssem0).wait()
        pltpu.make_async_copy(src_hbm.at[pl.ds(0, _E_PER_TILE)],
                              srcflat, ssem0).wait()
        pltpu.make_async_copy(dst2d_hbm.at[pl.ds(0, _NBLK)], dstbuf,
                              ssem0).wait()
        plsc.subcore_barrier()

        def gather_grp(grp, b):
            for kk in range(_GRP):
                idx = srcflat.at[pl.ds((grp * _GRP + kk) * _EBLK, _EBLK)]
                pltpu.async_copy(h_hbm.at[idx],
                                 rows[b].at[pl.ds(kk * _EBLK, _EBLK)],
                                 gsems[b])

        def buf_wait(b, sem):
            pltpu.make_async_copy(
                zero_hbm.at[pl.ds(0, rows_per_buf)], rows[b], sem).wait()

        def scat_grp(grp, b):
            for kk in range(_GRP):
                pltpu.async_copy(rows[b].at[pl.ds(kk * _EBLK, _EBLK)],
                                 accS.at[dstbuf.at[grp * _GRP + kk]],
                                 ssems[b], add=True)

        for b in range(_NBUF):
            gather_grp(b, b)

        def slot(grp, b, prefetch):
            buf_wait(b, gsems[b])
            scat_grp(grp, b)
            buf_wait(b, ssems[b])
            if prefetch:
                gather_grp(jnp.minimum(grp + _NBUF, _NGRP - 1), b)

        def body(g, carry):
            for b in range(_NBUF):
                slot(_NBUF * g + b, b, True)
            return carry

        lax.fori_loop(0, n_loop, body, 0)
        for b in range(n_tail):
            slot(n_loop * _NBUF + b, b, False)
        for b in range(n_tail, _NBUF):
            buf_wait(b, gsems[b])
        plsc.subcore_barrier()
        pltpu.sync_copy(accS.at[pl.ds(r0, _RPT)],
                        out_hbm.at[c, pl.ds(r0, _RPT)])

    return k


def _dinv(cnt_blk):
    return lax.rsqrt(1.0 + cnt_blk[0][:, 0:1] + cnt_blk[1][:, 0:1])


def _k1a_body(x_ref, w_ref, o_ref):
    o_ref[...] = jnp.dot(x_ref[...], w_ref[...],
                         preferred_element_type=jnp.float32)


def _k1b_body(g_ref, cnt_ref, o_ref):
    o_ref[...] = g_ref[...] * _dinv(cnt_ref)


def _k2_body(acc_ref, h_ref, cnt_ref, b_ref, o_ref):
    dinv = _dinv(cnt_ref)
    t = jnp.maximum(dinv * (acc_ref[0] + acc_ref[1] + h_ref[...]) + b_ref[...],
                    0.0)
    o_ref[...] = t * dinv


def _k3_body(acc_ref, t_ref, cnt_ref, w2_ref, b_ref, batch_ref,
             wfc_ref, bfc_ref, o_ref, pooled, counts):
    i = pl.program_id(0)

    @pl.when(i == 0)
    def _():
        pooled[...] = jnp.zeros_like(pooled)
        counts[...] = jnp.zeros_like(counts)

    dinv = _dinv(cnt_ref)
    u = dinv * (acc_ref[0] + acc_ref[1] + t_ref[...])
    out2 = jnp.maximum(
        jnp.dot(u, w2_ref[...], preferred_element_type=jnp.float32)
        + b_ref[...], 0.0)
    oh = (batch_ref[...] == lax.broadcasted_iota(jnp.int32, (1, _B), 1)
          ).astype(jnp.float32)
    cdims = (((0,), (0,)), ((), ()))
    pooled[...] += lax.dot_general(oh, out2, cdims,
                                   preferred_element_type=jnp.float32)
    counts[...] += lax.dot_general(oh, jnp.ones((_R, 1), jnp.float32), cdims,
                                   preferred_element_type=jnp.float32)

    @pl.when(i == _GRID - 1)
    def _():
        pm = pooled[...] / jnp.maximum(counts[...], 1.0)
        o_ref[...] = jnp.maximum(
            jnp.dot(pm, wfc_ref[...], preferred_element_type=jnp.float32)
            + bfc_ref[...], 0.0)


def _row_spec(d):
    return pl.BlockSpec((_R, d), lambda i: (i, 0))


def _full_spec(shape):
    return pl.BlockSpec(shape, lambda i: tuple(0 for _ in shape))


def _cnt_spec():
    return pl.BlockSpec((_NC, _R, _CW), lambda i: (0, i, 0))


def _acc_spec(d):
    return pl.BlockSpec((_NC, _R, d), lambda i: (0, i, 0))


@functools.lru_cache(maxsize=None)
def _k1a_call():
    return pl.pallas_call(
        _k1a_body,
        grid=(_GRID,),
        in_specs=[_row_spec(_D_IN), _full_spec((_D_IN, _H1))],
        out_specs=_row_spec(_H1),
        out_shape=jax.ShapeDtypeStruct((_N, _H1), jnp.float32),
    )


@functools.lru_cache(maxsize=None)
def _k1b_call():
    return pl.pallas_call(
        _k1b_body,
        grid=(_GRID,),
        in_specs=[_row_spec(_H1), _cnt_spec()],
        out_specs=_row_spec(_H1),
        out_shape=jax.ShapeDtypeStruct((_N, _H1), jnp.float32),
    )


@functools.lru_cache(maxsize=None)
def _k2_call():
    return pl.pallas_call(
        _k2_body,
        grid=(_GRID,),
        in_specs=[_acc_spec(_H1), _row_spec(_H1), _cnt_spec(),
                  _full_spec((1, _H1))],
        out_specs=_row_spec(_H1),
        out_shape=jax.ShapeDtypeStruct((_N, _H1), jnp.float32),
    )


@functools.lru_cache(maxsize=None)
def _k3_call():
    return pl.pallas_call(
        _k3_body,
        grid=(_GRID,),
        in_specs=[_acc_spec(_H1), _row_spec(_H1), _cnt_spec(),
                  _full_spec((_H1, _H2)), _full_spec((1, _H2)), _row_spec(1),
                  _full_spec((_H2, _D_OUT)), _full_spec((1, _D_OUT))],
        out_specs=_full_spec((_B, _D_OUT)),
        out_shape=jax.ShapeDtypeStruct((_B, _D_OUT), jnp.float32),
        scratch_shapes=[pltpu.VMEM((_B, _H2), jnp.float32),
                        pltpu.VMEM((_B, 1), jnp.float32)],
    )


def kernel(x, edge_index, batch, W1, b1, W2, b2, Wfc, bfc):
    f32 = jnp.float32
    i32 = jnp.int32
    pad_src = jnp.arange(_E_PAD - _E, dtype=i32) % _N
    src_pad = jnp.concatenate([edge_index[0].astype(i32), pad_src])
    pad_dst = _N + jnp.arange(_E_PAD - _E, dtype=i32) % (_NPAD - _N)
    dst_pad = jnp.concatenate([edge_index[1].astype(i32), pad_dst])
    dst2d = dst_pad.reshape(_E_PAD // _EBLK, _EBLK)
    batch2d = batch.astype(i32)[:, None]
    onesrow = jnp.tile(
        (jnp.arange(_CW) == 0).astype(f32)[None, :], (_EBLK, 1))
    z_deg = jnp.zeros((_NPAD, _CW), f32)
    z1 = jnp.zeros((_NPAD, _H1), f32)

    scat = _edge_scatter_kernel(_H1)
    cnt = _deg_kernel()(dst2d, onesrow, z_deg)
    g1 = _k1a_call()(x, W1)
    h1 = _k1b_call()(g1, cnt)
    acc1 = scat(src_pad, dst2d, h1, z1)
    t = _k2_call()(acc1, h1, cnt, b1.reshape(1, _H1))
    acc2 = scat(src_pad, dst2d, t, z1)
    out = _k3_call()(acc2, t, cnt, W2, b2.reshape(1, _H2),
                     batch2d, Wfc, bfc.reshape(1, _D_OUT))
    return out
